# Initial kernel scaffold; baseline (speedup 1.0000x reference)
import jax
import jax.numpy as jnp
from jax.experimental import pallas as pl


def kernel(h, c, row, col, batch, Wz_root, bz, Wz_rel, Wi_root, bi, Wi_rel, Wf_root, bf, Wf_rel, Wo_root, bo, Wo_rel):
    raise NotImplementedError("write your pallas kernel here")


# speedup vs baseline: 5.0852x; 5.0852x over previous
import functools

import jax
import jax.numpy as jnp
from jax import lax
from jax.experimental import pallas as pl
from jax.experimental.pallas import tpu as pltpu
from jax.experimental.pallas import tpu_sc as plsc

N = 10000
D = 128
E = 320000

NC = 2
NS = 16
NW = NC * NS
CH = 128
NCHUNK = -(-E // (NW * CH))
E_PAD = NW * NCHUNK * CH
AGG_ROWS = 10240
ZROWS = AGG_ROWS // NS


def _sc_segment_sum(h, row3, col3, zeros):
    mesh = plsc.VectorSubcoreMesh(core_axis_name="c", subcore_axis_name="s")

    @functools.partial(
        pl.kernel,
        mesh=mesh,
        out_type=jax.ShapeDtypeStruct((NC, AGG_ROWS, D), jnp.float32),
        scratch_types=[
            pltpu.VMEM((NCHUNK, CH), jnp.int32),
            pltpu.VMEM((NCHUNK, CH), jnp.int32),
            pltpu.VMEM((CH, D), jnp.float32),
            pltpu.VMEM_SHARED((AGG_ROWS, D), jnp.float32),
        ],
    )
    def sc_kernel(h_hbm, row_hbm, col_hbm, z_hbm, out_hbm,
                  rowidx_v, colidx_v, rows_v, agg_sp):
        c = lax.axis_index("c")
        s = lax.axis_index("s")
        wid = c * NS + s
        pltpu.sync_copy(row_hbm.at[wid], rowidx_v)
        pltpu.sync_copy(col_hbm.at[wid], colidx_v)
        pltpu.sync_copy(z_hbm, agg_sp.at[pl.ds(s * ZROWS, ZROWS)])
        plsc.subcore_barrier()

        def body(j, carry):
            pltpu.sync_copy(h_hbm.at[rowidx_v.at[j]], rows_v)
            pltpu.sync_copy(rows_v, agg_sp.at[colidx_v.at[j]], add=True)
            return carry

        lax.fori_loop(0, NCHUNK, body, 0)
        plsc.subcore_barrier()
        pltpu.sync_copy(agg_sp.at[pl.ds(s * ZROWS, ZROWS)],
                        out_hbm.at[c, pl.ds(s * ZROWS, ZROWS)])

    return sc_kernel(h, row3, col3, zeros)


def _dense_body(p_ref, h_ref, c_ref, wr_ref, wt_ref, b_ref, hn_ref, cn_ref):
    agg = p_ref[0] + p_ref[1]
    g = (jnp.dot(agg, wr_ref[...], preferred_element_type=jnp.float32)
         + jnp.dot(h_ref[...], wt_ref[...], preferred_element_type=jnp.float32)
         + b_ref[...])
    z = jnp.tanh(g[:, 0:D])
    i = jax.nn.sigmoid(g[:, D:2 * D])
    f = jax.nn.sigmoid(g[:, 2 * D:3 * D])
    o = jax.nn.sigmoid(g[:, 3 * D:4 * D])
    cn = f * c_ref[...] + i * z
    cn_ref[...] = cn
    hn_ref[...] = o * jnp.tanh(cn)


def _dense(partials, h, c, w_rel, w_root, b):
    blk = 1000
    grid = N // blk
    return pl.pallas_call(
        _dense_body,
        grid=(grid,),
        in_specs=[
            pl.BlockSpec((NC, blk, D), lambda n: (0, n, 0)),
            pl.BlockSpec((blk, D), lambda n: (n, 0)),
            pl.BlockSpec((blk, D), lambda n: (n, 0)),
            pl.BlockSpec((D, 4 * D), lambda n: (0, 0)),
            pl.BlockSpec((D, 4 * D), lambda n: (0, 0)),
            pl.BlockSpec((1, 4 * D), lambda n: (0, 0)),
        ],
        out_specs=[
            pl.BlockSpec((blk, D), lambda n: (n, 0)),
            pl.BlockSpec((blk, D), lambda n: (n, 0)),
        ],
        out_shape=[
            jax.ShapeDtypeStruct((N, D), jnp.float32),
            jax.ShapeDtypeStruct((N, D), jnp.float32),
        ],
    )(partials, h, c, w_rel, w_root, b)


def kernel(h, c, row, col, batch, Wz_root, bz, Wz_rel, Wi_root, bi, Wi_rel,
           Wf_root, bf, Wf_rel, Wo_root, bo, Wo_rel):
    pad = E_PAD - E
    row_p = jnp.concatenate([row, jnp.zeros((pad,), jnp.int32)])
    col_p = jnp.concatenate([col, jnp.full((pad,), N, jnp.int32)])
    row3 = row_p.reshape(NW, NCHUNK, CH)
    col3 = col_p.reshape(NW, NCHUNK, CH)
    zeros = jnp.zeros((ZROWS, D), jnp.float32)

    partials = _sc_segment_sum(h, row3, col3, zeros)

    w_rel = jnp.concatenate(
        [Wz_rel.T, Wi_rel.T, Wf_rel.T, Wo_rel.T], axis=1)
    w_root = jnp.concatenate(
        [Wz_root.T, Wi_root.T, Wf_root.T, Wo_root.T], axis=1)
    b = jnp.concatenate([bz, bi, bf, bo]).reshape(1, 4 * D)

    h_new, c_new = _dense(partials, h, c, w_rel, w_root, b)
    return (h_new, c_new)
